# Initial kernel scaffold; baseline (speedup 1.0000x reference)
import jax
import jax.numpy as jnp
from jax.experimental import pallas as pl


def kernel(trip_index, edge_type, params):
    raise NotImplementedError("write your pallas kernel here")


# speedup vs baseline: 3.6121x; 3.6121x over previous
import functools

import jax
import jax.numpy as jnp
from jax import lax
from jax.experimental import pallas as pl
from jax.experimental.pallas import tpu as pltpu
from jax.experimental.pallas import tpu_sc as plsc

N_NODES = 10000
N_EDGES = 320000
D = 128
NUM_RELS = 200

_NC = 2
_NS = 16
_NW = _NC * _NS
_EW = N_EDGES // _NW
_CHUNK = 80
_NSTEP = _EW // _CHUNK
_N_PAD = 10240
_ROWS_PER_TILE = _N_PAD // _NS
_MSG_BLK = 4000


def _tables_body(emb_h, w_h, rel_wt_p, emb_e_p, w_e, tn_out, tr_out):
    tn_out[...] = jnp.dot(emb_h[...], w_h[...], preferred_element_type=jnp.float32)
    re = jnp.dot(rel_wt_p[...], emb_e_p[...], preferred_element_type=jnp.float32)
    tr_out[...] = jnp.dot(re, w_e[...], preferred_element_type=jnp.float32)


_tables_call = pl.pallas_call(
    _tables_body,
    out_shape=[
        jax.ShapeDtypeStruct((N_NODES, D), jnp.float32),
        jax.ShapeDtypeStruct((NUM_RELS, D), jnp.float32),
    ],
)


def _sc_gather_body(tn_hbm, tr_hbm, src_hbm, rel_hbm, s_out,
                    src_idx, rel_idx, srows, rrows, sem_a, sem_b):
    c = lax.axis_index("c")
    s = lax.axis_index("s")
    wid = s * _NC + c
    base = wid * _EW

    def _step(g, carry):
        off = base + g * _CHUNK
        pltpu.sync_copy(src_hbm.at[pl.ds(off, _CHUNK)], src_idx)
        pltpu.sync_copy(rel_hbm.at[pl.ds(off, _CHUNK)], rel_idx)
        cp_a = pltpu.async_copy(tn_hbm.at[src_idx], srows, sem_a)
        cp_b = pltpu.async_copy(tr_hbm.at[rel_idx], rrows, sem_b)
        cp_a.wait()
        cp_b.wait()

        def _row(i, carry2):
            for j in range(D // 16):
                sl = pl.ds(j * 16, 16)
                srows[i, sl] = srows[i, sl] + rrows[i, sl]
            return carry2

        lax.fori_loop(0, _CHUNK, _row, 0)
        pltpu.sync_copy(srows, s_out.at[pl.ds(off, _CHUNK)])
        return carry

    lax.fori_loop(0, _NSTEP, _step, 0)


_sc_gather_call = functools.partial(
    pl.kernel,
    mesh=plsc.VectorSubcoreMesh(core_axis_name="c", subcore_axis_name="s"),
    out_type=[jax.ShapeDtypeStruct((N_EDGES, D), jnp.float32)],
    scratch_types=[
        pltpu.VMEM((_CHUNK,), jnp.int32),
        pltpu.VMEM((_CHUNK,), jnp.int32),
        pltpu.VMEM((_CHUNK, D), jnp.float32),
        pltpu.VMEM((_CHUNK, D), jnp.float32),
        pltpu.SemaphoreType.DMA,
        pltpu.SemaphoreType.DMA,
    ],
)(_sc_gather_body)


def _msg_body(s_ref, w_ref, b_ref, o_ref):
    o_ref[...] = jnp.maximum(
        jnp.dot(s_ref[...], w_ref[...], preferred_element_type=jnp.float32)
        + b_ref[...],
        0.0,
    )


_msg_call = pl.pallas_call(
    _msg_body,
    grid=(N_EDGES // _MSG_BLK,),
    in_specs=[
        pl.BlockSpec((_MSG_BLK, D), lambda i: (i, 0)),
        pl.BlockSpec((D, D), lambda i: (0, 0)),
        pl.BlockSpec((1, D), lambda i: (0, 0)),
    ],
    out_specs=pl.BlockSpec((_MSG_BLK, D), lambda i: (i, 0)),
    out_shape=jax.ShapeDtypeStruct((N_EDGES, D), jnp.float32),
)


def _sc_scatter_body(msg_hbm, dst_hbm, agg_out, cnt_out,
                     dst_idx, mrows, cnt_local, agg_sh, sem_a):
    c = lax.axis_index("c")
    s = lax.axis_index("s")
    row0 = s * _ROWS_PER_TILE
    n_sub = _ROWS_PER_TILE // _CHUNK
    out_row0 = c * _N_PAD + row0
    lanes = lax.iota(jnp.int32, 16)

    def _zero_body(i, carry):
        for j in range(D // 16):
            mrows[i, pl.ds(j * 16, 16)] = jnp.zeros((16,), jnp.float32)
        return carry

    lax.fori_loop(0, _CHUNK, _zero_body, 0)

    def _zero_cnt(i, carry):
        cnt_local[pl.ds(i * 16, 16)] = jnp.zeros((16,), jnp.float32)
        return carry

    lax.fori_loop(0, _N_PAD // 16, _zero_cnt, 0)

    def _fill_idx(buf, base):
        def _g(j, carry):
            buf[pl.ds(j * 16, 16)] = base + j * 16 + lanes
            return carry

        lax.fori_loop(0, _CHUNK // 16, _g, 0)

    def _zcopy(k, carry):
        _fill_idx(dst_idx, row0 + k * _CHUNK)
        pltpu.sync_copy(mrows, agg_sh.at[dst_idx])
        return carry

    lax.fori_loop(0, n_sub, _zcopy, 0)

    plsc.subcore_barrier()

    wid = s * _NC + c
    base = wid * _EW
    e0 = jnp.where(lanes == 0, 1.0, 0.0).astype(jnp.float32)

    def _step(g, carry):
        off = base + g * _CHUNK
        pltpu.sync_copy(dst_hbm.at[pl.ds(off, _CHUNK)], dst_idx)
        cp = pltpu.async_copy(msg_hbm.at[pl.ds(off, _CHUNK)], mrows, sem_a)

        def _cnt_blk(ib, carry2):
            dvec = dst_idx[pl.ds(ib * 16, 16)]
            for l in range(16):
                d = dvec[l]
                w = pl.ds(d, 16)
                cnt_local[w] = cnt_local[w] + e0
            return carry2

        lax.fori_loop(0, _CHUNK // 16, _cnt_blk, 0)

        cp.wait()
        pltpu.sync_copy(mrows, agg_sh.at[dst_idx], add=True)
        return carry

    lax.fori_loop(0, _NSTEP, _step, 0)

    plsc.subcore_barrier()

    def _pub(k, carry):
        _fill_idx(dst_idx, row0 + k * _CHUNK)
        pltpu.async_copy(agg_sh.at[dst_idx], mrows, sem_a).wait()
        pltpu.sync_copy(mrows, agg_out.at[pl.ds(out_row0 + k * _CHUNK, _CHUNK)])
        return carry

    lax.fori_loop(0, n_sub, _pub, 0)

    pltpu.sync_copy(cnt_local, cnt_out.at[wid])


_sc_scatter_call = functools.partial(
    pl.kernel,
    mesh=plsc.VectorSubcoreMesh(core_axis_name="c", subcore_axis_name="s"),
    out_type=[
        jax.ShapeDtypeStruct((_NC * _N_PAD, D), jnp.float32),
        jax.ShapeDtypeStruct((_NW, _N_PAD), jnp.float32),
    ],
    scratch_types=[
        pltpu.VMEM((_CHUNK,), jnp.int32),
        pltpu.VMEM((_CHUNK, D), jnp.float32),
        pltpu.VMEM((_N_PAD,), jnp.float32),
        pltpu.VMEM_SHARED((_N_PAD, D), jnp.float32),
        pltpu.SemaphoreType.DMA,
    ],
)(_sc_scatter_body)


def _bn(x, g, b):
    mu = jnp.mean(x, axis=0, keepdims=True)
    var = jnp.mean((x - mu) ** 2, axis=0, keepdims=True)
    return (x - mu) / jnp.sqrt(var + 1e-5) * g + b


def _tail_body(agg_part, cnt_part, w_op, b_op, bn_op_g, bn_op_b,
               w_concat, b_concat, bn_cell_g, bn_cell_b, bn_final_g, bn_final_b,
               w_c1, b_c1, w_c2, b_c2, w_c3, b_c3, out):
    agg = agg_part[0:N_NODES, :] + agg_part[_N_PAD:_N_PAD + N_NODES, :]
    cnt = jnp.sum(cnt_part[...], axis=0)[0:N_NODES].reshape(N_NODES, 1)
    h = agg / jnp.maximum(cnt, 1.0)
    h = jnp.dot(h, w_op[...], preferred_element_type=jnp.float32) + b_op[...]
    h = _bn(h, bn_op_g[...], bn_op_b[...])
    h = jnp.maximum(h, 0.0)
    h = jnp.dot(h, w_concat[...], preferred_element_type=jnp.float32) + b_concat[...]
    h = _bn(h, bn_cell_g[...], bn_cell_b[...])
    h = jnp.maximum(h, 0.0)
    h = _bn(h, bn_final_g[...], bn_final_b[...])
    h = jnp.maximum(h, 0.0)
    y = jnp.maximum(jnp.dot(h, w_c1[...], preferred_element_type=jnp.float32) + b_c1[...], 0.0)
    y = jnp.maximum(jnp.dot(y, w_c2[...], preferred_element_type=jnp.float32) + b_c2[...], 0.0)
    out[...] = jnp.dot(y, w_c3[...], preferred_element_type=jnp.float32) + b_c3[...]


_tail_call = pl.pallas_call(
    _tail_body,
    out_shape=jax.ShapeDtypeStruct((N_NODES, D), jnp.float32),
)


def _pad2(a, rows, cols):
    return jnp.zeros((rows, cols), a.dtype).at[: a.shape[0], : a.shape[1]].set(a)


def kernel(trip_index, edge_type, params):
    p = params
    src = jnp.asarray(trip_index[:, 1], jnp.int32)
    dst = jnp.asarray(trip_index[:, 2], jnp.int32)
    rel = jnp.asarray(edge_type, jnp.int32)

    rel_wt_p = _pad2(p["rel_wt"], NUM_RELS, D)
    emb_e_p = _pad2(p["emb_e"], D, D)
    tn, tr = _tables_call(p["emb_h"], p["W_h_init"], rel_wt_p, emb_e_p, p["W_e_init"])

    s_edges, = _sc_gather_call(tn, tr, src, rel)
    msg = _msg_call(s_edges, p["W_agg"], p["b_agg"].reshape(1, D))
    agg_part, cnt_part = _sc_scatter_call(msg, dst)

    w_c1 = _pad2(p["W_c1"], D, D)
    b_c1 = _pad2(p["b_c1"].reshape(1, -1), 1, D)
    w_c2 = _pad2(p["W_c2"], D, D)
    b_c2 = _pad2(p["b_c2"].reshape(1, -1), 1, D)
    w_c3 = _pad2(p["W_c3"], D, D)
    b_c3 = _pad2(p["b_c3"].reshape(1, -1), 1, D)

    logits_p = _tail_call(
        agg_part, cnt_part, p["W_op"], p["b_op"].reshape(1, D),
        p["bn_op_g"].reshape(1, D), p["bn_op_b"].reshape(1, D),
        p["W_concat"], p["b_concat"].reshape(1, D),
        p["bn_cell_g"].reshape(1, D), p["bn_cell_b"].reshape(1, D),
        p["bn_final_g"].reshape(1, D), p["bn_final_b"].reshape(1, D),
        w_c1, b_c1, w_c2, b_c2, w_c3, b_c3,
    )
    return logits_p[:, :100]
